# baseline (device time: 550925 ns/iter reference)
import functools

import jax
import jax.numpy as jnp
from jax import lax
from jax.experimental import pallas as pl
from jax.experimental.pallas import tpu as pltpu

B = 32
NB = 256
BS = 32
H = 16
D = 128
HD = H * D
P_LOCAL = 256
C = 8
MAXC = -(-NB // C)
NB_PAD = MAXC * C
G = B * MAXC
NSLOT = 4
PD = 3


def _compact(vals, valid):
    n = valid.shape[0]
    fv = valid.astype(jnp.float32)
    dest = jnp.cumsum(fv) - 1.0
    iota = lax.broadcasted_iota(jnp.float32, (n, n), 1)
    onehot = jnp.where((dest[:, None] == iota) & valid[:, None], 1.0, 0.0)
    return jnp.einsum("kj,jd->kd", vals * fv[None, :], onehot)


def kernel(Q, K, V, bt, lens):
    my_y = lax.axis_index("y")
    start = my_y * P_LOCAL

    pos = lax.broadcasted_iota(jnp.int32, (B, NB), 1)
    valid = pos < lens[:, None]
    local = valid & (bt >= start) & (bt < start + P_LOCAL)
    local_f = local.astype(jnp.float32)
    dest = jnp.cumsum(local_f, axis=1) - 1.0
    d_iota = lax.broadcasted_iota(jnp.float32, (1, 1, NB), 2)
    onehot = jnp.where((dest[:, :, None] == d_iota) & local[:, :, None], 1.0, 0.0)
    vals = (bt - start).astype(jnp.float32) * local_f
    loc = jnp.einsum("bj,bjd->bd", vals, onehot).astype(jnp.int32)
    loc = jnp.pad(loc, ((0, 0), (0, NB_PAD - NB)))
    cnt = jnp.sum(local.astype(jnp.int32), axis=1)

    nchunks = (cnt + C - 1) // C
    tgrid = lax.broadcasted_iota(jnp.int32, (B, MAXC), 1)
    bgrid = lax.broadcasted_iota(jnp.int32, (B, MAXC), 0)
    cvalid = (tgrid < nchunks[:, None]).reshape(-1)
    bt_flat = jnp.stack(
        [bgrid.reshape(-1).astype(jnp.float32), tgrid.reshape(-1).astype(jnp.float32)]
    )
    chunk_bt = _compact(bt_flat, cvalid).astype(jnp.int32)
    n_total = jnp.sum(nchunks).astype(jnp.int32).reshape(1)

    q_scaled = (Q[:, 0].reshape(B, 1, HD) * (D ** -0.5)).astype(jnp.bfloat16)
    KV16 = jnp.stack(
        [K.astype(jnp.bfloat16), V.astype(jnp.bfloat16)], axis=1
    ).reshape(P_LOCAL, 2, BS, HD)

    def body(loc_ref, cnt_ref, cbt_ref, nt_ref, q_ref, kv_ref,
             num_ref, lout_ref,
             acc_ref, l_ref, racc_ref, rl_ref,
             kvbuf, dma_sems, send_sems, recv_sems):
        my_x = lax.axis_index("x")
        yy = lax.axis_index("y")
        my_z = lax.axis_index("z")
        partner = (my_x, 1 - yy, my_z)
        n_tot = nt_ref[0]

        barrier = pltpu.get_barrier_semaphore()
        pl.semaphore_signal(barrier, inc=1, device_id=partner,
                            device_id_type=pl.DeviceIdType.MESH)
        pl.semaphore_wait(barrier, 1)

        acc_ref[...] = jnp.zeros((B, 1, HD), jnp.float32)
        l_ref[...] = jnp.zeros((B, H), jnp.float32)

        def issue_chunk(g):
            slot = lax.rem(g, NSLOT)
            b = cbt_ref[0, g]
            t = cbt_ref[1, g]
            for c in range(C):
                page = loc_ref[b, t * C + c]
                pltpu.make_async_copy(
                    kv_ref.at[page], kvbuf.at[slot, c], dma_sems.at[slot, c]
                ).start()

        def wait_chunk(slot):
            for c in range(C):
                pltpu.make_async_copy(
                    kv_ref.at[0], kvbuf.at[slot, c], dma_sems.at[slot, c]
                ).wait()

        for r in range(PD):
            @pl.when(r < n_tot)
            def _():
                issue_chunk(r)

        def gbody(g, _):
            slot = lax.rem(g, NSLOT)

            @pl.when(g + PD < n_tot)
            def _():
                issue_chunk(g + PD)

            wait_chunk(slot)

            b = cbt_ref[0, g]
            t = cbt_ref[1, g]
            hcol = lax.broadcasted_iota(jnp.int32, (H, HD), 1) // D
            hrow = lax.broadcasted_iota(jnp.int32, (H, HD), 0)
            blkf = jnp.where(hcol == hrow, 1.0, 0.0)
            qb = q_ref[pl.ds(b, 1)].reshape(1, HD)
            qblkT = (jnp.broadcast_to(qb.astype(jnp.float32), (H, HD))
                     * blkf).astype(jnp.bfloat16)
            kc = kvbuf[slot, :, 0].reshape(C * BS, HD)
            vc = kvbuf[slot, :, 1].reshape(C * BS, HD)
            s = lax.dot_general(kc, qblkT, (((1,), (1,)), ((), ())),
                                preferred_element_type=jnp.float32)
            row = lax.broadcasted_iota(jnp.int32, (C * BS, H), 0)
            p = jnp.where(row // BS + t * C < cnt_ref[b], jnp.exp(s), 0.0)
            l_ref[pl.ds(b, 1)] += jnp.sum(p, axis=0)[None]
            pv = lax.dot_general(p.astype(jnp.bfloat16), vc,
                                 (((0,), (0,)), ((), ())),
                                 preferred_element_type=jnp.float32)
            acc_ref[pl.ds(b, 1)] += jnp.sum(
                pv * blkf, axis=0, keepdims=True)[None]
            return 0

        lax.fori_loop(0, n_tot, gbody, 0)

        copies = [
            pltpu.make_async_remote_copy(
                src_ref=src, dst_ref=dst,
                send_sem=send_sems.at[n], recv_sem=recv_sems.at[n],
                device_id=partner, device_id_type=pl.DeviceIdType.MESH)
            for n, (src, dst) in enumerate(
                [(acc_ref, racc_ref), (l_ref, rl_ref)])
        ]
        for cp in copies:
            cp.start()
        for cp in copies:
            cp.wait()

        num_ref[...] = acc_ref[...] + racc_ref[...]
        lout_ref[...] = l_ref[...] + rl_ref[...]

        @functools.partial(pl.run_scoped, exit_sem=pltpu.SemaphoreType.REGULAR)
        def _(exit_sem):
            pl.semaphore_signal(exit_sem, inc=1, device_id=partner,
                                device_id_type=pl.DeviceIdType.MESH)
            pl.semaphore_wait(exit_sem, 1)

    num, l_tot = pl.pallas_call(
        body,
        out_shape=[
            jax.ShapeDtypeStruct((B, 1, HD), jnp.float32),
            jax.ShapeDtypeStruct((B, H), jnp.float32),
        ],
        in_specs=[
            pl.BlockSpec(memory_space=pltpu.SMEM),
            pl.BlockSpec(memory_space=pltpu.SMEM),
            pl.BlockSpec(memory_space=pltpu.SMEM),
            pl.BlockSpec(memory_space=pltpu.SMEM),
            pl.BlockSpec(memory_space=pltpu.VMEM),
            pl.BlockSpec(memory_space=pl.ANY),
        ],
        out_specs=[
            pl.BlockSpec(memory_space=pltpu.VMEM),
            pl.BlockSpec(memory_space=pltpu.VMEM),
        ],
        scratch_shapes=[
            pltpu.VMEM((B, 1, HD), jnp.float32),
            pltpu.VMEM((B, H), jnp.float32),
            pltpu.VMEM((B, 1, HD), jnp.float32),
            pltpu.VMEM((B, H), jnp.float32),
            pltpu.VMEM((NSLOT, C, 2, BS, HD), jnp.bfloat16),
            pltpu.SemaphoreType.DMA((NSLOT, C)),
            pltpu.SemaphoreType.DMA((2,)),
            pltpu.SemaphoreType.DMA((2,)),
        ],
        compiler_params=pltpu.CompilerParams(collective_id=0),
    )(loc, cnt, chunk_bt, n_total, q_scaled, KV16)
    return (num.reshape(B, H, D) / l_tot[:, :, None])[:, None]


# device time: 412007 ns/iter; 1.3372x vs baseline; 1.3372x over previous
import functools

import jax
import jax.numpy as jnp
from jax import lax
from jax.experimental import pallas as pl
from jax.experimental.pallas import tpu as pltpu

B = 32
NB = 256
BS = 32
H = 16
D = 128
P_LOCAL = 256
C = 8
MAXC = NB // C
G = B * MAXC
NSLOT = 4
PD = 3


def _compact(vals, valid):
    n = valid.shape[0]
    fv = valid.astype(jnp.float32)
    dest = jnp.cumsum(fv) - 1.0
    iota = lax.broadcasted_iota(jnp.float32, (n, n), 1)
    onehot = jnp.where((dest[:, None] == iota) & valid[:, None], 1.0, 0.0)
    return jnp.einsum("kj,jd->kd", vals * fv[None, :], onehot)


def kernel(Q, K, V, bt, lens):
    my_y = lax.axis_index("y")
    start = my_y * P_LOCAL

    pos = lax.broadcasted_iota(jnp.int32, (B, NB), 1)
    valid = pos < lens[:, None]
    local = valid & (bt >= start) & (bt < start + P_LOCAL)
    local_f = local.astype(jnp.float32)
    dest = jnp.cumsum(local_f, axis=1) - 1.0
    d_iota = lax.broadcasted_iota(jnp.float32, (1, 1, NB), 2)
    onehot = jnp.where((dest[:, :, None] == d_iota) & local[:, :, None], 1.0, 0.0)
    vals = (bt - start).astype(jnp.float32) * local_f
    loc = jnp.einsum("bj,bjd->bd", vals, onehot).astype(jnp.int32)
    cnt = jnp.sum(local.astype(jnp.int32), axis=1)

    nchunks = (cnt + C - 1) // C
    tgrid = lax.broadcasted_iota(jnp.int32, (B, MAXC), 1)
    bgrid = lax.broadcasted_iota(jnp.int32, (B, MAXC), 0)
    cvalid = (tgrid < nchunks[:, None]).reshape(-1)
    bt_flat = jnp.stack(
        [bgrid.reshape(-1).astype(jnp.float32), tgrid.reshape(-1).astype(jnp.float32)]
    )
    chunk_bt = _compact(bt_flat, cvalid).astype(jnp.int32)
    n_total = jnp.sum(nchunks).astype(jnp.int32).reshape(1)

    q_scaled = Q[:, 0] * (D ** -0.5)

    def body(loc_ref, cnt_ref, cbt_ref, nt_ref, q_ref, k_ref, v_ref, out_ref,
             acc_ref, l_ref, racc_ref, rl_ref,
             kbuf, vbuf, dma_sems, send_sems, recv_sems):
        my_x = lax.axis_index("x")
        yy = lax.axis_index("y")
        my_z = lax.axis_index("z")
        partner = (my_x, 1 - yy, my_z)
        n_tot = nt_ref[0]

        barrier = pltpu.get_barrier_semaphore()
        pl.semaphore_signal(barrier, inc=1, device_id=partner,
                            device_id_type=pl.DeviceIdType.MESH)
        pl.semaphore_wait(barrier, 1)

        acc_ref[...] = jnp.zeros((B, H, D), jnp.float32)
        l_ref[...] = jnp.zeros((B, H), jnp.float32)

        def chunk_dmas(g, slot):
            b = cbt_ref[0, g]
            t = cbt_ref[1, g]
            for c in range(C):
                page = loc_ref[b, t * C + c]
                yield pltpu.make_async_copy(
                    k_ref.at[page], kbuf.at[slot, c], dma_sems.at[slot, c, 0])
                yield pltpu.make_async_copy(
                    v_ref.at[page], vbuf.at[slot, c], dma_sems.at[slot, c, 1])

        def issue_chunk(g):
            for dma in chunk_dmas(g, lax.rem(g, NSLOT)):
                dma.start()

        for r in range(PD):
            @pl.when(r < n_tot)
            def _():
                issue_chunk(r)

        def gbody(g, _):
            slot = lax.rem(g, NSLOT)

            @pl.when(g + PD < n_tot)
            def _():
                issue_chunk(g + PD)

            for dma in chunk_dmas(g, slot):
                dma.wait()

            b = cbt_ref[0, g]
            t = cbt_ref[1, g]
            q = q_ref[pl.ds(b, 1)]
            kc = kbuf[slot].reshape(C * BS, H, D)
            vc = vbuf[slot].reshape(C * BS, H, D)
            s = jnp.sum(kc * q, axis=-1)
            row = lax.broadcasted_iota(jnp.int32, (C * BS, H), 0)
            p = jnp.where(row // BS + t * C < cnt_ref[b], jnp.exp(s), 0.0)
            l_ref[pl.ds(b, 1)] += jnp.sum(p, axis=0)[None]
            acc_ref[pl.ds(b, 1)] += jnp.sum(p[:, :, None] * vc, axis=0)[None]
            return 0

        lax.fori_loop(0, n_tot, gbody, 0)

        copies = [
            pltpu.make_async_remote_copy(
                src_ref=src, dst_ref=dst,
                send_sem=send_sems.at[n], recv_sem=recv_sems.at[n],
                device_id=partner, device_id_type=pl.DeviceIdType.MESH)
            for n, (src, dst) in enumerate(
                [(acc_ref, racc_ref), (l_ref, rl_ref)])
        ]
        for c in copies:
            c.start()
        for c in copies:
            c.wait()

        l_tot = l_ref[...] + rl_ref[...]
        num = acc_ref[...] + racc_ref[...]
        out_ref[...] = num / l_tot[:, :, None]

        @functools.partial(pl.run_scoped, exit_sem=pltpu.SemaphoreType.REGULAR)
        def _(exit_sem):
            pl.semaphore_signal(exit_sem, inc=1, device_id=partner,
                                device_id_type=pl.DeviceIdType.MESH)
            pl.semaphore_wait(exit_sem, 1)

    out = pl.pallas_call(
        body,
        out_shape=jax.ShapeDtypeStruct((B, H, D), jnp.float32),
        in_specs=[
            pl.BlockSpec(memory_space=pltpu.SMEM),
            pl.BlockSpec(memory_space=pltpu.SMEM),
            pl.BlockSpec(memory_space=pltpu.SMEM),
            pl.BlockSpec(memory_space=pltpu.SMEM),
            pl.BlockSpec(memory_space=pltpu.VMEM),
            pl.BlockSpec(memory_space=pl.ANY),
            pl.BlockSpec(memory_space=pl.ANY),
        ],
        out_specs=pl.BlockSpec(memory_space=pltpu.VMEM),
        scratch_shapes=[
            pltpu.VMEM((B, H, D), jnp.float32),
            pltpu.VMEM((B, H), jnp.float32),
            pltpu.VMEM((B, H, D), jnp.float32),
            pltpu.VMEM((B, H), jnp.float32),
            pltpu.VMEM((NSLOT, C, BS, H, D), jnp.float32),
            pltpu.VMEM((NSLOT, C, BS, H, D), jnp.float32),
            pltpu.SemaphoreType.DMA((NSLOT, C, 2)),
            pltpu.SemaphoreType.DMA((2,)),
            pltpu.SemaphoreType.DMA((2,)),
        ],
        compiler_params=pltpu.CompilerParams(collective_id=0),
    )(loc, cnt, chunk_bt, n_total, q_scaled, K, V)
    return out[:, None]


# device time: 90823 ns/iter; 6.0659x vs baseline; 4.5364x over previous
import functools

import jax
import jax.numpy as jnp
from jax import lax
from jax.experimental import pallas as pl
from jax.experimental.pallas import tpu as pltpu

B = 32
NB = 256
BS = 32
H = 16
D = 128
P_LOCAL = 256
C = 8
MAXC = NB // C
G = B * MAXC
NSLOT = 4
PD = 3
NREP = 8
NROUND = 4


def _compact(vals, valid):
    n = valid.shape[0]
    fv = valid.astype(jnp.float32)
    dest = jnp.cumsum(fv) - 1.0
    iota = lax.broadcasted_iota(jnp.float32, (n, n), 1)
    onehot = jnp.where((dest[:, None] == iota) & valid[:, None], 1.0, 0.0)
    return jnp.einsum("kj,jd->kd", vals * fv[None, :], onehot)


def kernel(Q, K, V, bt, lens):
    my_y = lax.axis_index("y")
    start = my_y * P_LOCAL

    pos = lax.broadcasted_iota(jnp.int32, (B, NB), 1)
    valid = pos < lens[:, None]
    local = valid & (bt >= start) & (bt < start + P_LOCAL)
    local_f = local.astype(jnp.float32)
    dest = jnp.cumsum(local_f, axis=1) - 1.0
    d_iota = lax.broadcasted_iota(jnp.float32, (1, 1, NB), 2)
    onehot = jnp.where((dest[:, :, None] == d_iota) & local[:, :, None], 1.0, 0.0)
    vals = (bt - start).astype(jnp.float32) * local_f
    loc = jnp.einsum("bj,bjd->bd", vals, onehot).astype(jnp.int32)
    cnt = jnp.sum(local.astype(jnp.int32), axis=1)

    nchunks = (cnt + C - 1) // C
    tgrid = lax.broadcasted_iota(jnp.int32, (B, MAXC), 1)
    bgrid = lax.broadcasted_iota(jnp.int32, (B, MAXC), 0)
    cvalid = (tgrid < nchunks[:, None]).reshape(-1)
    bt_flat = jnp.stack(
        [bgrid.reshape(-1).astype(jnp.float32), tgrid.reshape(-1).astype(jnp.float32)]
    )
    chunk_bt = _compact(bt_flat, cvalid).astype(jnp.int32)
    n_total = jnp.sum(nchunks).astype(jnp.int32).reshape(1)

    q_scaled = Q[:, 0] * (D ** -0.5)

    def body(loc_ref, cnt_ref, cbt_ref, nt_ref, q_ref, k_ref, v_ref, out_ref,
             acc_ref, l_ref, racc_ref, rl_ref,
             kbuf, vbuf, dma_sems, send_sems, recv_sems):
        my_x = lax.axis_index("x")
        yy = lax.axis_index("y")
        my_z = lax.axis_index("z")
        n_tot = nt_ref[0]
        rep = my_x * 4 + my_z

        partners = [
            (my_x, 1 - yy, my_z),
            (1 - my_x, yy, my_z),
            (my_x, yy, my_z ^ 1),
            (my_x, yy, my_z ^ 2),
        ]

        barrier = pltpu.get_barrier_semaphore()
        for pt in partners:
            pl.semaphore_signal(barrier, inc=1, device_id=pt,
                                device_id_type=pl.DeviceIdType.MESH)
        pl.semaphore_wait(barrier, NROUND)

        acc_ref[...] = jnp.zeros((B, H, D), jnp.float32)
        l_ref[...] = jnp.zeros((B, H), jnp.float32)

        def chunk_dmas(g, slot):
            b = cbt_ref[0, g]
            t = cbt_ref[1, g]
            for c in range(C):
                page = loc_ref[b, t * C + c]
                yield pltpu.make_async_copy(
                    k_ref.at[page], kbuf.at[slot, c], dma_sems.at[slot, c, 0])
                yield pltpu.make_async_copy(
                    v_ref.at[page], vbuf.at[slot, c], dma_sems.at[slot, c, 1])

        def issue_chunk(j):
            for dma in chunk_dmas(rep + NREP * j, lax.rem(j, NSLOT)):
                dma.start()

        n_mine = jnp.maximum(0, (n_tot - rep + NREP - 1) // NREP)

        for r in range(PD):
            @pl.when(r < n_mine)
            def _():
                issue_chunk(r)

        def gbody(j, _):
            slot = lax.rem(j, NSLOT)

            @pl.when(j + PD < n_mine)
            def _():
                issue_chunk(j + PD)

            g = rep + NREP * j
            for dma in chunk_dmas(g, slot):
                dma.wait()

            b = cbt_ref[0, g]
            t = cbt_ref[1, g]
            q = q_ref[pl.ds(b, 1)]
            kc = kbuf[slot].reshape(C * BS, H, D)
            vc = vbuf[slot].reshape(C * BS, H, D)
            s = jnp.sum(kc * q, axis=-1)
            row = lax.broadcasted_iota(jnp.int32, (C * BS, H), 0)
            p = jnp.where(row // BS + t * C < cnt_ref[b], jnp.exp(s), 0.0)
            l_ref[pl.ds(b, 1)] += jnp.sum(p, axis=0)[None]
            acc_ref[pl.ds(b, 1)] += jnp.sum(p[:, :, None] * vc, axis=0)[None]
            return 0

        lax.fori_loop(0, n_mine, gbody, 0)

        for rnd, pt in enumerate(partners):
            ca = pltpu.make_async_remote_copy(
                src_ref=acc_ref, dst_ref=racc_ref.at[rnd],
                send_sem=send_sems.at[rnd, 0], recv_sem=recv_sems.at[rnd, 0],
                device_id=pt, device_id_type=pl.DeviceIdType.MESH)
            cl = pltpu.make_async_remote_copy(
                src_ref=l_ref, dst_ref=rl_ref.at[rnd],
                send_sem=send_sems.at[rnd, 1], recv_sem=recv_sems.at[rnd, 1],
                device_id=pt, device_id_type=pl.DeviceIdType.MESH)
            ca.start()
            cl.start()
            ca.wait()
            cl.wait()
            acc_ref[...] += racc_ref[rnd]
            l_ref[...] += rl_ref[rnd]

        out_ref[...] = acc_ref[...] / l_ref[...][:, :, None]

        @functools.partial(pl.run_scoped, exit_sem=pltpu.SemaphoreType.REGULAR)
        def _(exit_sem):
            for pt in partners:
                pl.semaphore_signal(exit_sem, inc=1, device_id=pt,
                                    device_id_type=pl.DeviceIdType.MESH)
            pl.semaphore_wait(exit_sem, NROUND)

    out = pl.pallas_call(
        body,
        out_shape=jax.ShapeDtypeStruct((B, H, D), jnp.float32),
        in_specs=[
            pl.BlockSpec(memory_space=pltpu.SMEM),
            pl.BlockSpec(memory_space=pltpu.SMEM),
            pl.BlockSpec(memory_space=pltpu.SMEM),
            pl.BlockSpec(memory_space=pltpu.SMEM),
            pl.BlockSpec(memory_space=pltpu.VMEM),
            pl.BlockSpec(memory_space=pl.ANY),
            pl.BlockSpec(memory_space=pl.ANY),
        ],
        out_specs=pl.BlockSpec(memory_space=pltpu.VMEM),
        scratch_shapes=[
            pltpu.VMEM((B, H, D), jnp.float32),
            pltpu.VMEM((B, H), jnp.float32),
            pltpu.VMEM((NROUND, B, H, D), jnp.float32),
            pltpu.VMEM((NROUND, B, H), jnp.float32),
            pltpu.VMEM((NSLOT, C, BS, H, D), jnp.float32),
            pltpu.VMEM((NSLOT, C, BS, H, D), jnp.float32),
            pltpu.SemaphoreType.DMA((NSLOT, C, 2)),
            pltpu.SemaphoreType.DMA((NROUND, 2)),
            pltpu.SemaphoreType.DMA((NROUND, 2)),
        ],
        compiler_params=pltpu.CompilerParams(collective_id=0),
    )(loc, cnt, chunk_bt, n_total, q_scaled, K, V)
    return out[:, None]
